# baseline (speedup 1.0000x reference)
import functools

import jax
import jax.numpy as jnp
from jax import lax
from jax.experimental import pallas as pl
from jax.experimental.pallas import tpu as pltpu
from jax.experimental.pallas import tpu_sc as plsc

_B, _T, _D = 4, 2048, 768
_N = _B * _T
_NC, _NS = 2, 16
_NW = _NC * _NS
_POS_PER_W = _T // _NW
_CHUNK = 16
_SUB = _POS_PER_W // _CHUNK
_NCHUNK = _B * _SUB
_LANES = 16
_NBUF = 4


def _make_embed_kernel():
    mesh = plsc.VectorSubcoreMesh(core_axis_name="c", subcore_axis_name="s")

    @functools.partial(
        pl.kernel,
        out_type=jax.ShapeDtypeStruct((_N, _D), jnp.float32),
        mesh=mesh,
        scratch_types=(
            [pltpu.VMEM((_POS_PER_W, _D), jnp.float32),
             pltpu.VMEM((_NCHUNK, _CHUNK), jnp.int32)]
            + [pltpu.VMEM((_CHUNK, _D), jnp.float32)] * _NBUF
            + [pltpu.SemaphoreType.DMA] * (2 + 2 * _NBUF)
        ),
    )
    def embed(idx_hbm, tok_hbm, pos_hbm, out_hbm, *scratch):
        pos_v, idx_v = scratch[0], scratch[1]
        rows = scratch[2:2 + _NBUF]
        sem_pos, sem_idx = scratch[2 + _NBUF], scratch[3 + _NBUF]
        sem_g = scratch[4 + _NBUF:4 + 2 * _NBUF]
        sem_o = scratch[4 + 2 * _NBUF:4 + 3 * _NBUF]

        wid = lax.axis_index("s") * _NC + lax.axis_index("c")
        p0 = wid * _POS_PER_W

        pos_cp = pltpu.async_copy(pos_hbm.at[pl.ds(p0, _POS_PER_W)],
                                  pos_v, sem_pos)

        def row_base(k):
            b, s = k // _SUB, k % _SUB
            return b * _T + p0 + s * _CHUNK

        idx_cps = [
            pltpu.async_copy(idx_hbm.at[pl.ds(row_base(k), _CHUNK)],
                             idx_v.at[k], sem_idx)
            for k in range(_NCHUNK)
        ]

        def start(k):
            buf = k % _NBUF
            return pltpu.async_copy(tok_hbm.at[idx_v.at[k]], rows[buf],
                                    sem_g[buf])

        for cp in idx_cps:
            cp.wait()
        ahead = _NBUF - 1
        h_g = {j: start(j) for j in range(min(ahead, _NCHUNK))}
        h_o = {}
        pos_cp.wait()
        for k in range(_NCHUNK):
            buf = k % _NBUF
            if k + ahead < _NCHUNK:
                if k - 1 >= 0:
                    h_o[k - 1].wait()
                h_g[k + ahead] = start(k + ahead)
            h_g[k].wait()
            pos_off = (k % _SUB) * _CHUNK

            @plsc.parallel_loop(0, _CHUNK)
            def _row(r):
                @plsc.parallel_loop(0, _D, step=_LANES, unroll=8)
                def _col(c):
                    plsc.addupdate(rows[buf].at[r, pl.ds(c, _LANES)],
                                   pos_v[pos_off + r, pl.ds(c, _LANES)])

            h_o[k] = pltpu.async_copy(
                rows[buf], out_hbm.at[pl.ds(row_base(k), _CHUNK)], sem_o[buf])
        for k in range(max(0, _NCHUNK - _NBUF), _NCHUNK):
            h_o[k].wait()

    return embed


_embed = _make_embed_kernel()


@jax.jit
def kernel(idx, tok_table, pos_table):
    idx_flat = idx.reshape(_N).astype(jnp.int32)
    out = _embed(idx_flat, tok_table, pos_table)
    return out.reshape(_B, _T, _D)


# speedup vs baseline: 1.0696x; 1.0696x over previous
import functools

import jax
import jax.numpy as jnp
from jax import lax
from jax.experimental import pallas as pl
from jax.experimental.pallas import tpu as pltpu
from jax.experimental.pallas import tpu_sc as plsc

_B, _T, _D = 4, 2048, 768
_N = _B * _T
_NC, _NS = 2, 16
_NW = _NC * _NS
_POS_PER_W = _T // _NW
_CHUNK = 32
_SUB = _POS_PER_W // _CHUNK
_NCHUNK = _B * _SUB
_LANES = 16
_NBUF = 3
_HALF = _CHUNK // 2


def _make_embed_kernel():
    mesh = plsc.VectorSubcoreMesh(core_axis_name="c", subcore_axis_name="s")

    @functools.partial(
        pl.kernel,
        out_type=jax.ShapeDtypeStruct((_N, _D), jnp.float32),
        mesh=mesh,
        scratch_types=(
            [pltpu.VMEM((_POS_PER_W, _D), jnp.float32),
             pltpu.VMEM((_NCHUNK, _CHUNK), jnp.int32)]
            + [pltpu.VMEM((_CHUNK, _D), jnp.float32)] * _NBUF
            + [pltpu.SemaphoreType.DMA] * (2 + 2 * _NBUF)
        ),
    )
    def embed(idx_hbm, tok_hbm, pos_hbm, out_hbm, *scratch):
        pos_v, idx_v = scratch[0], scratch[1]
        rows = scratch[2:2 + _NBUF]
        sem_pos, sem_idx = scratch[2 + _NBUF], scratch[3 + _NBUF]
        sem_g = scratch[4 + _NBUF:4 + 2 * _NBUF]
        sem_o = scratch[4 + 2 * _NBUF:4 + 3 * _NBUF]

        wid = lax.axis_index("s") * _NC + lax.axis_index("c")
        p0 = wid * _POS_PER_W

        pos_cp = pltpu.async_copy(pos_hbm.at[pl.ds(p0, _POS_PER_W)],
                                  pos_v, sem_pos)

        def row_base(k):
            b, s = k // _SUB, k % _SUB
            return b * _T + p0 + s * _CHUNK

        idx_cps = [
            pltpu.async_copy(idx_hbm.at[pl.ds(row_base(k), _CHUNK)],
                             idx_v.at[k], sem_idx)
            for k in range(_NCHUNK)
        ]

        def start(k):
            buf = k % _NBUF
            return pltpu.async_copy(tok_hbm.at[idx_v.at[k]], rows[buf],
                                    sem_g[buf])

        for cp in idx_cps:
            cp.wait()
        ahead = _NBUF - 1
        h_g = {j: start(j) for j in range(min(ahead, _NCHUNK))}
        h_o = {}
        pos_cp.wait()
        for k in range(_NCHUNK):
            buf = k % _NBUF
            if k + ahead < _NCHUNK:
                if k - 1 >= 0:
                    for cp in h_o[k - 1]:
                        cp.wait()
                h_g[k + ahead] = start(k + ahead)
            h_g[k].wait()
            pos_off = (k % _SUB) * _CHUNK

            halves = []
            for h in range(2):
                h0 = h * _HALF

                @plsc.parallel_loop(h0, h0 + _HALF)
                def _row(r):
                    @plsc.parallel_loop(0, _D, step=_LANES, unroll=8)
                    def _col(c):
                        plsc.addupdate(rows[buf].at[r, pl.ds(c, _LANES)],
                                       pos_v[pos_off + r, pl.ds(c, _LANES)])

                halves.append(pltpu.async_copy(
                    rows[buf].at[pl.ds(h0, _HALF)],
                    out_hbm.at[pl.ds(row_base(k) + h0, _HALF)], sem_o[buf]))
            h_o[k] = halves
        for k in range(max(0, _NCHUNK - _NBUF), _NCHUNK):
            for cp in h_o[k]:
                cp.wait()

    return embed


_embed = _make_embed_kernel()


@jax.jit
def kernel(idx, tok_table, pos_table):
    idx_flat = idx.reshape(_N).astype(jnp.int32)
    out = _embed(idx_flat, tok_table, pos_table)
    return out.reshape(_B, _T, _D)


# speedup vs baseline: 1.0749x; 1.0049x over previous
import functools

import jax
import jax.numpy as jnp
from jax import lax
from jax.experimental import pallas as pl
from jax.experimental.pallas import tpu as pltpu
from jax.experimental.pallas import tpu_sc as plsc

_B, _T, _D = 4, 2048, 768
_N = _B * _T
_NC, _NS = 2, 16
_NW = _NC * _NS
_POS_PER_W = _T // _NW
_CHUNK = 32
_SUB = _POS_PER_W // _CHUNK
_NCHUNK = _B * _SUB
_LANES = 16
_NBUF = 3
_HALF = _CHUNK // 2


def _make_embed_kernel():
    mesh = plsc.VectorSubcoreMesh(core_axis_name="c", subcore_axis_name="s")

    @functools.partial(
        pl.kernel,
        out_type=jax.ShapeDtypeStruct((_B, _T, _D), jnp.float32),
        mesh=mesh,
        scratch_types=(
            [pltpu.VMEM((_POS_PER_W, _D), jnp.float32),
             pltpu.VMEM((_NCHUNK, _CHUNK), jnp.int32)]
            + [pltpu.VMEM((_CHUNK, _D), jnp.float32)] * _NBUF
            + [pltpu.SemaphoreType.DMA] * (2 + 2 * _NBUF)
        ),
    )
    def embed(idx_hbm, tok_hbm, pos_hbm, out_hbm, *scratch):
        pos_v, idx_v = scratch[0], scratch[1]
        rows = scratch[2:2 + _NBUF]
        sem_pos, sem_idx = scratch[2 + _NBUF], scratch[3 + _NBUF]
        sem_g = scratch[4 + _NBUF:4 + 2 * _NBUF]
        sem_o = scratch[4 + 2 * _NBUF:4 + 3 * _NBUF]

        wid = lax.axis_index("s") * _NC + lax.axis_index("c")
        p0 = wid * _POS_PER_W

        pos_cp = pltpu.async_copy(pos_hbm.at[pl.ds(p0, _POS_PER_W)],
                                  pos_v, sem_pos)

        def bt(k):
            b, s = k // _SUB, k % _SUB
            return b, p0 + s * _CHUNK

        idx_cps = [
            pltpu.async_copy(
                idx_hbm.at[bt(k)[0], pl.ds(bt(k)[1], _CHUNK)],
                idx_v.at[k], sem_idx)
            for k in range(_NCHUNK)
        ]

        def start(k):
            buf = k % _NBUF
            return pltpu.async_copy(tok_hbm.at[idx_v.at[k]], rows[buf],
                                    sem_g[buf])

        for cp in idx_cps:
            cp.wait()
        ahead = _NBUF - 1
        h_g = {j: start(j) for j in range(min(ahead, _NCHUNK))}
        h_o = {}
        pos_cp.wait()
        for k in range(_NCHUNK):
            buf = k % _NBUF
            if k + ahead < _NCHUNK:
                if k - 1 >= 0:
                    for cp in h_o[k - 1]:
                        cp.wait()
                h_g[k + ahead] = start(k + ahead)
            h_g[k].wait()
            pos_off = (k % _SUB) * _CHUNK

            halves = []
            for h in range(2):
                h0 = h * _HALF

                @plsc.parallel_loop(h0, h0 + _HALF)
                def _row(r):
                    @plsc.parallel_loop(0, _D, step=_LANES, unroll=8)
                    def _col(c):
                        plsc.addupdate(rows[buf].at[r, pl.ds(c, _LANES)],
                                       pos_v[pos_off + r, pl.ds(c, _LANES)])

                b, t0 = bt(k)
                halves.append(pltpu.async_copy(
                    rows[buf].at[pl.ds(h0, _HALF)],
                    out_hbm.at[b, pl.ds(t0 + h0, _HALF)], sem_o[buf]))
            h_o[k] = halves
        for k in range(max(0, _NCHUNK - _NBUF), _NCHUNK):
            for cp in h_o[k]:
                cp.wait()

    return embed


_embed = _make_embed_kernel()


@jax.jit
def kernel(idx, tok_table, pos_table):
    return _embed(idx.astype(jnp.int32), tok_table, pos_table)


# speedup vs baseline: 1.0991x; 1.0225x over previous
import functools

import jax
import jax.numpy as jnp
from jax import lax
from jax.experimental import pallas as pl
from jax.experimental.pallas import tpu as pltpu
from jax.experimental.pallas import tpu_sc as plsc

_B, _T, _D = 4, 2048, 768
_NC, _NS = 2, 16
_NW = _NC * _NS
_POS_PER_W = _T // _NW
_CHUNK = 32
_SUB = _POS_PER_W // _CHUNK
_NCHUNK = _B * _SUB
_LANES = 16
_HALF = _CHUNK // 2


def _make_embed_kernel():
    mesh = plsc.VectorSubcoreMesh(core_axis_name="c", subcore_axis_name="s")

    @functools.partial(
        pl.kernel,
        out_type=jax.ShapeDtypeStruct((_B, _T, _D), jnp.float32),
        mesh=mesh,
        scratch_types=[
            pltpu.VMEM((_POS_PER_W, _D), jnp.float32),
            pltpu.VMEM((_NCHUNK, _CHUNK), jnp.int32),
            pltpu.VMEM((_CHUNK, _D), jnp.float32),
            pltpu.VMEM((_CHUNK, _D), jnp.float32),
            pltpu.SemaphoreType.DMA,
            pltpu.SemaphoreType.DMA,
            pltpu.SemaphoreType.DMA,
            pltpu.SemaphoreType.DMA,
            pltpu.SemaphoreType.DMA,
            pltpu.SemaphoreType.DMA,
        ],
    )
    def embed(idx_hbm, tok_hbm, pos_hbm, out_hbm,
              pos_v, idx_v, rows0, rows1,
              sem_pos, sem_idx, sem_g0, sem_g1, sem_o0, sem_o1):
        wid = lax.axis_index("s") * _NC + lax.axis_index("c")
        p0 = wid * _POS_PER_W
        rows = (rows0, rows1)
        sem_g = (sem_g0, sem_g1)
        sem_o = (sem_o0, sem_o1)

        pos_cp = pltpu.async_copy(pos_hbm.at[pl.ds(p0, _POS_PER_W)],
                                  pos_v, sem_pos)

        idx_cps = [
            pltpu.async_copy(
                idx_hbm.at[k // _SUB,
                           pl.ds(p0 + (k % _SUB) * _CHUNK, _CHUNK)],
                idx_v.at[k], sem_idx)
            for k in range(_NCHUNK)
        ]

        def gather(k, buf):
            return pltpu.make_async_copy(tok_hbm.at[idx_v.at[k]], rows[buf],
                                         sem_g[buf])

        def out_half(b, buf, h0):
            return pltpu.make_async_copy(
                rows[buf].at[pl.ds(h0, _HALF)],
                out_hbm.at[b, pl.ds(p0 + buf * _CHUNK + h0, _HALF)],
                sem_o[buf])

        def drain_out(b, buf):
            pltpu.make_async_copy(
                rows[buf],
                out_hbm.at[b, pl.ds(p0 + buf * _CHUNK, _CHUNK)],
                sem_o[buf]).wait()

        def add_half(buf, h0):
            pos_base = buf * _CHUNK

            @plsc.parallel_loop(h0, h0 + _HALF)
            def _row(r):
                @plsc.parallel_loop(0, _D, step=_LANES, unroll=8)
                def _col(c):
                    plsc.addupdate(rows[buf].at[r, pl.ds(c, _LANES)],
                                   pos_v[pos_base + r, pl.ds(c, _LANES)])

        def add_and_out(b, buf, between=None):
            add_half(buf, 0)
            out_half(b, buf, 0).start()
            if between is not None:
                between()
            add_half(buf, _HALF)
            out_half(b, buf, _HALF).start()

        for cp in idx_cps:
            cp.wait()
        gather(0, 0).start()
        pos_cp.wait()

        @pl.loop(0, _B)
        def _batch(j):
            c0 = _SUB * j

            @pl.when(j > 0)
            def _():
                drain_out(j - 1, 1)
            gather(c0 + 1, 1).start()
            gather(c0, 0).wait()
            add_and_out(j, 0)
            gather(c0 + 1, 1).wait()

            def _refill_buf0():
                @pl.when(j + 1 < _B)
                def _():
                    drain_out(j, 0)
                    gather(c0 + _SUB, 0).start()

            add_and_out(j, 1, between=_refill_buf0)

        drain_out(_B - 1, 0)
        drain_out(_B - 1, 1)

    return embed


_embed = _make_embed_kernel()


@jax.jit
def kernel(idx, tok_table, pos_table):
    return _embed(idx.astype(jnp.int32), tok_table, pos_table)
